# baseline (speedup 1.0000x reference)
import functools

import jax
import jax.numpy as jnp
from jax import lax
from jax.experimental import pallas as pl
from jax.experimental.pallas import tpu as pltpu
from jax.experimental.pallas import tpu_sc as plsc

B = 16384
INPUT_DIM = 128
EMBED_DIM = 16
NUM_CHAMPS = 100000
NCHUNK = 782
TROWS = NCHUNK * 128

NC = 2
NS = 16
NW = NC * NS
BPW = B // NW
GCH = 128
NGC = BPW // GCH
GSZ = 8
NGRP = 4


def _sc_lin_body(tt_hbm, out_hbm,
                 sa0, sa1, sa2, sa3, sa4, sa5, sa6, sa7,
                 sb0, sb1, sb2, sb3, sb4, sb5, sb6, sb7,
                 oa, ob, ssem_a, ssem_b, wsem_a, wsem_b):
    wid = lax.axis_index("s") * NC + lax.axis_index("c")
    stage = ((sa0, sa1, sa2, sa3, sa4, sa5, sa6, sa7),
             (sb0, sb1, sb2, sb3, sb4, sb5, sb6, sb7))
    outv = (oa, ob)
    ssem = (ssem_a, ssem_b)
    wsem = (wsem_a, wsem_b)
    lanes = lax.iota(jnp.int32, 16)

    def chunk_id(g, i):
        return lax.rem(GSZ * (NGRP * wid + g) + i, NCHUNK)

    def fire_stages(g, s):
        for i in range(GSZ):
            c0 = pl.multiple_of(128 * chunk_id(g, i), 128)
            pltpu.async_copy(tt_hbm.at[:, pl.ds(c0, 128)], stage[s][i],
                             ssem[s])

    def drain_stages(s):
        for i in range(GSZ):
            pltpu.make_async_copy(tt_hbm.at[:, pl.ds(0, 128)], stage[s][i],
                                  ssem[s]).wait()

    def fire_writes(g, s):
        for i in range(GSZ):
            w0 = pl.multiple_of(2048 * chunk_id(g, i), 8)
            pltpu.async_copy(outv[s].at[pl.ds(2048 * i, 2048)],
                             out_hbm.at[pl.ds(w0, 2048)], wsem[s])

    def drain_writes(s):
        pltpu.make_async_copy(out_hbm.at[pl.ds(0, GSZ * 2048)], outv[s],
                              wsem[s]).wait()

    def transpose_group(s):
        for i in range(GSZ):
            def tbody(j, carry):
                col = jnp.full((16,), j, jnp.int32)
                outv[s][pl.ds(2048 * i + 16 * j, 16)] = plsc.load_gather(
                    stage[s][i], [lanes, col])
                return carry

            lax.fori_loop(0, 128, tbody, 0, unroll=4)

    fire_stages(0, 0)
    fire_stages(1, 1)
    for g in range(NGRP):
        s = g % 2
        drain_stages(s)
        if g >= 2:
            drain_writes(s)
        transpose_group(s)
        fire_writes(g, s)
        if g + 2 < NGRP:
            fire_stages(g + 2, s)
    drain_writes(0)
    drain_writes(1)


_sc_lin = functools.partial(
    pl.kernel,
    mesh=plsc.VectorSubcoreMesh(core_axis_name="c", subcore_axis_name="s",
                                num_cores=NC, num_subcores=NS),
    compiler_params=pltpu.CompilerParams(needs_layout_passes=False),
    out_type=jax.ShapeDtypeStruct((TROWS * EMBED_DIM,), jnp.float32),
    scratch_types=(
        [pltpu.VMEM((16, 128), jnp.float32) for _ in range(16)]
        + [pltpu.VMEM((GSZ * 2048,), jnp.float32) for _ in range(2)]
        + [pltpu.SemaphoreType.DMA for _ in range(4)]
    ),
)(_sc_lin_body)


def _sc_prod_body(table_hbm, idx1_hbm, idx2_hbm, out_hbm,
                  idx1_v, idx2_v, prod_v,
                  r1a, r1b, r2a, r2b,
                  sem1a, sem1b, sem2a, sem2b):
    wid = lax.axis_index("s") * NC + lax.axis_index("c")

    a = wid // 2
    b0 = 4 * (wid % 2)
    pltpu.sync_copy(idx1_hbm.at[a, pl.ds(b0, NGC)], idx1_v)
    pltpu.sync_copy(idx2_hbm.at[a, pl.ds(b0, NGC)], idx2_v)

    r1 = (r1a, r1b)
    r2 = (r2a, r2b)
    sems1 = (sem1a, sem1b)
    sems2 = (sem2a, sem2b)
    cps = {}

    def fire(k):
        b = k % 2
        cps[(1, k)] = pltpu.async_copy(table_hbm.at[idx1_v.at[k]], r1[b], sems1[b])
        cps[(2, k)] = pltpu.async_copy(table_hbm.at[idx2_v.at[k]], r2[b], sems2[b])

    def consume(k):
        b = k % 2
        cps.pop((1, k)).wait()
        cps.pop((2, k)).wait()
        for n in range(GCH):
            g = k * GCH + n
            e1 = r1[b][n, :]
            e2 = r2[b][n, :]
            prod_v[g // 64, (g // 8) % 8, pl.ds(16 * (g % 8), 16)] = e1 * e2

    fire(0)
    fire(1)
    for k in range(NGC):
        consume(k)
        if k + 2 < NGC:
            fire(k + 2)

    pltpu.sync_copy(prod_v, out_hbm.at[pl.ds(8 * wid, 8)])


_sc_prod = functools.partial(
    pl.kernel,
    mesh=plsc.VectorSubcoreMesh(core_axis_name="c", subcore_axis_name="s",
                                num_cores=NC, num_subcores=NS),
    compiler_params=pltpu.CompilerParams(use_tc_tiling_on_sc=False),
    out_type=jax.ShapeDtypeStruct((B // 64, 8, 128), jnp.float32),
    scratch_types=[
        pltpu.VMEM((NGC, GCH), jnp.int32),
        pltpu.VMEM((NGC, GCH), jnp.int32),
        pltpu.VMEM((BPW // 64, 8, 128), jnp.float32),
        pltpu.VMEM((GCH, EMBED_DIM), jnp.float32),
        pltpu.VMEM((GCH, EMBED_DIM), jnp.float32),
        pltpu.VMEM((GCH, EMBED_DIM), jnp.float32),
        pltpu.VMEM((GCH, EMBED_DIM), jnp.float32),
        pltpu.SemaphoreType.DMA,
        pltpu.SemaphoreType.DMA,
        pltpu.SemaphoreType.DMA,
        pltpu.SemaphoreType.DMA,
    ],
)(_sc_prod_body)


TC_BLK = 2048
TC_NBLK = B // TC_BLK


def _tc_feat_body(x1_ref, x2_ref, w_ref, out_ref):
    d = x1_ref[...] - x2_ref[...]
    out_ref[...] = lax.dot_general(w_ref[...], d, (((1,), (1,)), ((), ())),
                                   preferred_element_type=jnp.float32)


_tc_feat = pl.pallas_call(
    _tc_feat_body,
    grid=(TC_NBLK,),
    in_specs=[
        pl.BlockSpec((TC_BLK, INPUT_DIM), lambda i: (i, 0)),
        pl.BlockSpec((TC_BLK, INPUT_DIM), lambda i: (i, 0)),
        pl.BlockSpec((1, INPUT_DIM), lambda i: (0, 0)),
    ],
    out_specs=pl.BlockSpec((1, TC_BLK), lambda i: (0, i)),
    out_shape=jax.ShapeDtypeStruct((1, B), jnp.float32),
)


def _tc_comb_body(feat_ref, prod_ref, scale_ref, out_ref):
    pr = prod_ref[...].reshape(TC_BLK // 8, 128)
    pr8 = jnp.broadcast_to(pr[:, None, :], (TC_BLK // 8, 8, 128))
    pr8 = pr8.reshape(TC_BLK, 128)
    si = lax.broadcasted_iota(jnp.int32, (TC_BLK, 128), 0)
    ci = lax.broadcasted_iota(jnp.int32, (TC_BLK, 128), 1)
    msk = (ci // EMBED_DIM == si % 8).astype(jnp.float32)
    ones = jnp.ones((1, 128), jnp.float32)
    inter = lax.dot_general(ones, pr8 * msk, (((1,), (1,)), ((), ())),
                            preferred_element_type=jnp.float32)
    out_ref[...] = feat_ref[...] + scale_ref[0, 0] * inter


_tc_comb = pl.pallas_call(
    _tc_comb_body,
    grid=(TC_NBLK,),
    in_specs=[
        pl.BlockSpec((1, TC_BLK), lambda i: (0, i)),
        pl.BlockSpec((TC_BLK // 64, 8, 128), lambda i: (i, 0, 0)),
        pl.BlockSpec(memory_space=pltpu.SMEM),
    ],
    out_specs=pl.BlockSpec((1, TC_BLK), lambda i: (0, i)),
    out_shape=jax.ShapeDtypeStruct((1, B), jnp.float32),
)


def kernel(x_1, x_2, idx_1, idx_2, W, table, scale):
    feat = _tc_feat(x_1, x_2, W)
    tbl_lin = _sc_lin(table.T).reshape(TROWS, EMBED_DIM)
    idx1 = idx_1.astype(jnp.int32).reshape(B // 1024, 8, 128)
    idx2 = idx_2.astype(jnp.int32).reshape(B // 1024, 8, 128)
    prod = _sc_prod(tbl_lin, idx1, idx2)
    scale2d = jnp.asarray(scale, jnp.float32).reshape(1, 1)
    out = _tc_comb(feat, prod, scale2d)
    return out.reshape(B, 1)


# speedup vs baseline: 1.5477x; 1.5477x over previous
import functools

import jax
import jax.numpy as jnp
from jax import lax
from jax.experimental import pallas as pl
from jax.experimental.pallas import tpu as pltpu
from jax.experimental.pallas import tpu_sc as plsc

B = 16384
INPUT_DIM = 128
EMBED_DIM = 16
NUM_CHAMPS = 100000
NCHUNK = 782
TROWS = NCHUNK * 128

NC = 2
NS = 16
NW = NC * NS
BPW = B // NW
GCH = 128
NGC = BPW // GCH
GSZ = 8
NGRP = 4


def _sc_lin_body(tt_hbm, out_hbm,
                 sa0, sa1, sa2, sa3, sa4, sa5, sa6, sa7,
                 sb0, sb1, sb2, sb3, sb4, sb5, sb6, sb7,
                 oa, ob, ssem_a, ssem_b, wsem_a, wsem_b):
    wid = lax.axis_index("s") * NC + lax.axis_index("c")
    stage = ((sa0, sa1, sa2, sa3, sa4, sa5, sa6, sa7),
             (sb0, sb1, sb2, sb3, sb4, sb5, sb6, sb7))
    outv = (oa, ob)
    ssem = (ssem_a, ssem_b)
    wsem = (wsem_a, wsem_b)
    lanes = lax.iota(jnp.int32, 16)

    def chunk_id(g, i):
        return lax.rem(GSZ * (NGRP * wid + g) + i, NCHUNK)

    def fire_stages(g, s):
        for i in range(GSZ):
            c0 = pl.multiple_of(128 * chunk_id(g, i), 128)
            pltpu.async_copy(tt_hbm.at[:, pl.ds(c0, 128)], stage[s][i],
                             ssem[s])

    def drain_stages(s):
        for i in range(GSZ):
            pltpu.make_async_copy(tt_hbm.at[:, pl.ds(0, 128)], stage[s][i],
                                  ssem[s]).wait()

    def fire_writes(g, s):
        for i in range(GSZ):
            w0 = pl.multiple_of(2048 * chunk_id(g, i), 8)
            pltpu.async_copy(outv[s].at[pl.ds(2048 * i, 2048)],
                             out_hbm.at[pl.ds(w0, 2048)], wsem[s])

    def drain_writes(s):
        pltpu.make_async_copy(out_hbm.at[pl.ds(0, GSZ * 2048)], outv[s],
                              wsem[s]).wait()

    def transpose_group(s):
        for i in range(GSZ):
            def tbody(k, carry):
                t = (k + lanes) & 127
                d = plsc.load_gather(stage[s][i], [lanes, t])
                pos = (t << 4) + lanes + (2048 * i)
                plsc.store_scatter(outv[s], [pos], d)
                return carry

            lax.fori_loop(0, 128, tbody, 0, unroll=4)

    fire_stages(0, 0)
    fire_stages(1, 1)
    for g in range(NGRP):
        s = g % 2
        drain_stages(s)
        if g >= 2:
            drain_writes(s)
        transpose_group(s)
        fire_writes(g, s)
        if g + 2 < NGRP:
            fire_stages(g + 2, s)
    drain_writes(0)
    drain_writes(1)


_sc_lin = functools.partial(
    pl.kernel,
    mesh=plsc.VectorSubcoreMesh(core_axis_name="c", subcore_axis_name="s",
                                num_cores=NC, num_subcores=NS),
    compiler_params=pltpu.CompilerParams(needs_layout_passes=False),
    out_type=jax.ShapeDtypeStruct((TROWS * EMBED_DIM,), jnp.float32),
    scratch_types=(
        [pltpu.VMEM((16, 128), jnp.float32) for _ in range(16)]
        + [pltpu.VMEM((GSZ * 2048,), jnp.float32) for _ in range(2)]
        + [pltpu.SemaphoreType.DMA for _ in range(4)]
    ),
)(_sc_lin_body)


def _sc_prod_body(table_hbm, idx1_hbm, idx2_hbm, out_hbm,
                  idx1_v, idx2_v, prod_v,
                  r1a, r1b, r2a, r2b,
                  sem1a, sem1b, sem2a, sem2b):
    wid = lax.axis_index("s") * NC + lax.axis_index("c")

    a = wid // 2
    b0 = 4 * (wid % 2)
    pltpu.sync_copy(idx1_hbm.at[a, pl.ds(b0, NGC)], idx1_v)
    pltpu.sync_copy(idx2_hbm.at[a, pl.ds(b0, NGC)], idx2_v)

    r1 = (r1a, r1b)
    r2 = (r2a, r2b)
    sems1 = (sem1a, sem1b)
    sems2 = (sem2a, sem2b)
    cps = {}

    def fire(k):
        b = k % 2
        cps[(1, k)] = pltpu.async_copy(table_hbm.at[idx1_v.at[k]], r1[b], sems1[b])
        cps[(2, k)] = pltpu.async_copy(table_hbm.at[idx2_v.at[k]], r2[b], sems2[b])

    def consume(k):
        b = k % 2
        cps.pop((1, k)).wait()
        cps.pop((2, k)).wait()
        for n in range(GCH):
            g = k * GCH + n
            e1 = r1[b][n, :]
            e2 = r2[b][n, :]
            prod_v[g // 64, (g // 8) % 8, pl.ds(16 * (g % 8), 16)] = e1 * e2

    fire(0)
    fire(1)
    for k in range(NGC):
        consume(k)
        if k + 2 < NGC:
            fire(k + 2)

    pltpu.sync_copy(prod_v, out_hbm.at[pl.ds(8 * wid, 8)])


_sc_prod = functools.partial(
    pl.kernel,
    mesh=plsc.VectorSubcoreMesh(core_axis_name="c", subcore_axis_name="s",
                                num_cores=NC, num_subcores=NS),
    compiler_params=pltpu.CompilerParams(use_tc_tiling_on_sc=False),
    out_type=jax.ShapeDtypeStruct((B // 64, 8, 128), jnp.float32),
    scratch_types=[
        pltpu.VMEM((NGC, GCH), jnp.int32),
        pltpu.VMEM((NGC, GCH), jnp.int32),
        pltpu.VMEM((BPW // 64, 8, 128), jnp.float32),
        pltpu.VMEM((GCH, EMBED_DIM), jnp.float32),
        pltpu.VMEM((GCH, EMBED_DIM), jnp.float32),
        pltpu.VMEM((GCH, EMBED_DIM), jnp.float32),
        pltpu.VMEM((GCH, EMBED_DIM), jnp.float32),
        pltpu.SemaphoreType.DMA,
        pltpu.SemaphoreType.DMA,
        pltpu.SemaphoreType.DMA,
        pltpu.SemaphoreType.DMA,
    ],
)(_sc_prod_body)


TC_BLK = 2048
TC_NBLK = B // TC_BLK


def _tc_feat_body(x1_ref, x2_ref, w_ref, out_ref):
    d = x1_ref[...] - x2_ref[...]
    out_ref[...] = lax.dot_general(w_ref[...], d, (((1,), (1,)), ((), ())),
                                   preferred_element_type=jnp.float32)


_tc_feat = pl.pallas_call(
    _tc_feat_body,
    grid=(TC_NBLK,),
    in_specs=[
        pl.BlockSpec((TC_BLK, INPUT_DIM), lambda i: (i, 0)),
        pl.BlockSpec((TC_BLK, INPUT_DIM), lambda i: (i, 0)),
        pl.BlockSpec((1, INPUT_DIM), lambda i: (0, 0)),
    ],
    out_specs=pl.BlockSpec((1, TC_BLK), lambda i: (0, i)),
    out_shape=jax.ShapeDtypeStruct((1, B), jnp.float32),
)


def _tc_comb_body(feat_ref, prod_ref, scale_ref, out_ref):
    pr = prod_ref[...].reshape(TC_BLK // 8, 128)
    pr8 = jnp.broadcast_to(pr[:, None, :], (TC_BLK // 8, 8, 128))
    pr8 = pr8.reshape(TC_BLK, 128)
    si = lax.broadcasted_iota(jnp.int32, (TC_BLK, 128), 0)
    ci = lax.broadcasted_iota(jnp.int32, (TC_BLK, 128), 1)
    msk = (ci // EMBED_DIM == si % 8).astype(jnp.float32)
    ones = jnp.ones((1, 128), jnp.float32)
    inter = lax.dot_general(ones, pr8 * msk, (((1,), (1,)), ((), ())),
                            preferred_element_type=jnp.float32)
    out_ref[...] = feat_ref[...] + scale_ref[0, 0] * inter


_tc_comb = pl.pallas_call(
    _tc_comb_body,
    grid=(TC_NBLK,),
    in_specs=[
        pl.BlockSpec((1, TC_BLK), lambda i: (0, i)),
        pl.BlockSpec((TC_BLK // 64, 8, 128), lambda i: (i, 0, 0)),
        pl.BlockSpec(memory_space=pltpu.SMEM),
    ],
    out_specs=pl.BlockSpec((1, TC_BLK), lambda i: (0, i)),
    out_shape=jax.ShapeDtypeStruct((1, B), jnp.float32),
)


def kernel(x_1, x_2, idx_1, idx_2, W, table, scale):
    feat = _tc_feat(x_1, x_2, W)
    tbl_lin = _sc_lin(table.T).reshape(TROWS, EMBED_DIM)
    idx1 = idx_1.astype(jnp.int32).reshape(B // 1024, 8, 128)
    idx2 = idx_2.astype(jnp.int32).reshape(B // 1024, 8, 128)
    prod = _sc_prod(tbl_lin, idx1, idx2)
    scale2d = jnp.asarray(scale, jnp.float32).reshape(1, 1)
    out = _tc_comb(feat, prod, scale2d)
    return out.reshape(B, 1)


# speedup vs baseline: 1.7787x; 1.1493x over previous
import functools

import jax
import jax.numpy as jnp
from jax import lax
from jax.experimental import pallas as pl
from jax.experimental.pallas import tpu as pltpu
from jax.experimental.pallas import tpu_sc as plsc

B = 16384
INPUT_DIM = 128
EMBED_DIM = 16
NUM_CHAMPS = 100000
NCHUNK = 782
TROWS = NCHUNK * 128

NC = 2
NS = 16
NW = NC * NS
BPW = B // NW
GCH = 128
NGC = BPW // GCH
GSZ = 5
NGRP = 5


def _sc_lin_body(tt_hbm, out_hbm,
                 sa0, sa1, sa2, sa3, sa4,
                 sb0, sb1, sb2, sb3, sb4,
                 oa, ob, ssem_a, ssem_b, wsem_a, wsem_b):
    wid = lax.axis_index("s") * NC + lax.axis_index("c")
    stage = ((sa0, sa1, sa2, sa3, sa4),
             (sb0, sb1, sb2, sb3, sb4))
    outv = (oa, ob)
    ssem = (ssem_a, ssem_b)
    wsem = (wsem_a, wsem_b)
    lanes = lax.iota(jnp.int32, 16)

    def chunk_id(g, i):
        return lax.rem(GSZ * (NGRP * wid + g) + i, NCHUNK)

    def fire_stages(g, s):
        for i in range(GSZ):
            c0 = pl.multiple_of(128 * chunk_id(g, i), 128)
            pltpu.async_copy(tt_hbm.at[:, pl.ds(c0, 128)], stage[s][i],
                             ssem[s])

    def drain_stages(s):
        for i in range(GSZ):
            pltpu.make_async_copy(tt_hbm.at[:, pl.ds(0, 128)], stage[s][i],
                                  ssem[s]).wait()

    def fire_writes(g, s):
        for i in range(GSZ):
            w0 = pl.multiple_of(2048 * chunk_id(g, i), 8)
            pltpu.async_copy(outv[s].at[pl.ds(2048 * i, 2048)],
                             out_hbm.at[pl.ds(w0, 2048)], wsem[s])

    def drain_writes(s):
        pltpu.make_async_copy(out_hbm.at[pl.ds(0, GSZ * 2048)], outv[s],
                              wsem[s]).wait()

    def transpose_group(s):
        for i in range(GSZ):
            def tbody(k, carry):
                t = (k + lanes) & 127
                d = plsc.load_gather(stage[s][i], [lanes, t])
                pos = (t << 4) + lanes + (2048 * i)
                plsc.store_scatter(outv[s], [pos], d)
                return carry

            lax.fori_loop(0, 128, tbody, 0, unroll=4)

    fire_stages(0, 0)
    fire_stages(1, 1)
    for g in range(NGRP):
        s = g % 2
        drain_stages(s)
        if g >= 2:
            drain_writes(s)
        transpose_group(s)
        fire_writes(g, s)
        if g + 2 < NGRP:
            fire_stages(g + 2, s)
    drain_writes(0)
    drain_writes(1)


_sc_lin = functools.partial(
    pl.kernel,
    mesh=plsc.VectorSubcoreMesh(core_axis_name="c", subcore_axis_name="s",
                                num_cores=NC, num_subcores=NS),
    compiler_params=pltpu.CompilerParams(needs_layout_passes=False),
    out_type=jax.ShapeDtypeStruct((TROWS * EMBED_DIM,), jnp.float32),
    scratch_types=(
        [pltpu.VMEM((16, 128), jnp.float32) for _ in range(2 * GSZ)]
        + [pltpu.VMEM((GSZ * 2048,), jnp.float32) for _ in range(2)]
        + [pltpu.SemaphoreType.DMA for _ in range(4)]
    ),
)(_sc_lin_body)


def _sc_prod_body(table_hbm, idx1_hbm, idx2_hbm, out_hbm,
                  idx1_v, idx2_v, prod_v,
                  r1a, r1b, r2a, r2b,
                  sem1a, sem1b, sem2a, sem2b):
    wid = lax.axis_index("s") * NC + lax.axis_index("c")

    a = wid // 2
    b0 = 4 * (wid % 2)
    pltpu.sync_copy(idx1_hbm.at[a, pl.ds(b0, NGC)], idx1_v)
    pltpu.sync_copy(idx2_hbm.at[a, pl.ds(b0, NGC)], idx2_v)

    r1 = (r1a, r1b)
    r2 = (r2a, r2b)
    sems1 = (sem1a, sem1b)
    sems2 = (sem2a, sem2b)
    cps = {}

    def fire(k):
        b = k % 2
        cps[(1, k)] = pltpu.async_copy(table_hbm.at[idx1_v.at[k]], r1[b], sems1[b])
        cps[(2, k)] = pltpu.async_copy(table_hbm.at[idx2_v.at[k]], r2[b], sems2[b])

    def consume(k):
        b = k % 2
        cps.pop((1, k)).wait()
        cps.pop((2, k)).wait()
        for n in range(GCH):
            g = k * GCH + n
            e1 = r1[b][n, :]
            e2 = r2[b][n, :]
            prod_v[g // 64, (g // 8) % 8, pl.ds(16 * (g % 8), 16)] = e1 * e2

    fire(0)
    fire(1)
    for k in range(NGC):
        consume(k)
        if k + 2 < NGC:
            fire(k + 2)

    pltpu.sync_copy(prod_v, out_hbm.at[pl.ds(8 * wid, 8)])


_sc_prod = functools.partial(
    pl.kernel,
    mesh=plsc.VectorSubcoreMesh(core_axis_name="c", subcore_axis_name="s",
                                num_cores=NC, num_subcores=NS),
    compiler_params=pltpu.CompilerParams(use_tc_tiling_on_sc=False),
    out_type=jax.ShapeDtypeStruct((B // 64, 8, 128), jnp.float32),
    scratch_types=[
        pltpu.VMEM((NGC, GCH), jnp.int32),
        pltpu.VMEM((NGC, GCH), jnp.int32),
        pltpu.VMEM((BPW // 64, 8, 128), jnp.float32),
        pltpu.VMEM((GCH, EMBED_DIM), jnp.float32),
        pltpu.VMEM((GCH, EMBED_DIM), jnp.float32),
        pltpu.VMEM((GCH, EMBED_DIM), jnp.float32),
        pltpu.VMEM((GCH, EMBED_DIM), jnp.float32),
        pltpu.SemaphoreType.DMA,
        pltpu.SemaphoreType.DMA,
        pltpu.SemaphoreType.DMA,
        pltpu.SemaphoreType.DMA,
    ],
)(_sc_prod_body)


TC_BLK = 2048
TC_NBLK = B // TC_BLK


def _tc_feat_body(x1_ref, x2_ref, w_ref, out_ref):
    d = x1_ref[...] - x2_ref[...]
    out_ref[...] = lax.dot_general(w_ref[...], d, (((1,), (1,)), ((), ())),
                                   preferred_element_type=jnp.float32)


_tc_feat = pl.pallas_call(
    _tc_feat_body,
    grid=(TC_NBLK,),
    in_specs=[
        pl.BlockSpec((TC_BLK, INPUT_DIM), lambda i: (i, 0)),
        pl.BlockSpec((TC_BLK, INPUT_DIM), lambda i: (i, 0)),
        pl.BlockSpec((1, INPUT_DIM), lambda i: (0, 0)),
    ],
    out_specs=pl.BlockSpec((1, TC_BLK), lambda i: (0, i)),
    out_shape=jax.ShapeDtypeStruct((1, B), jnp.float32),
)


TC_CBLK = 8192
TC_CNBLK = B // TC_CBLK


def _tc_comb_body(feat_ref, prod_ref, scale_ref, out_ref):
    pr = prod_ref[...].reshape(TC_CBLK // 8, 128)
    pr8 = jnp.broadcast_to(pr[:, None, :], (TC_CBLK // 8, 8, 128))
    pr8 = pr8.reshape(TC_CBLK, 128)
    si = lax.broadcasted_iota(jnp.int32, (TC_CBLK, 128), 0)
    ci = lax.broadcasted_iota(jnp.int32, (TC_CBLK, 128), 1)
    msk = (ci // EMBED_DIM == si % 8).astype(jnp.float32)
    ones = jnp.ones((1, 128), jnp.float32)
    inter = lax.dot_general(ones, pr8 * msk, (((1,), (1,)), ((), ())),
                            preferred_element_type=jnp.float32)
    out_ref[...] = feat_ref[...] + scale_ref[0, 0] * inter


_tc_comb = pl.pallas_call(
    _tc_comb_body,
    grid=(TC_CNBLK,),
    in_specs=[
        pl.BlockSpec((1, TC_CBLK), lambda i: (0, i)),
        pl.BlockSpec((TC_CBLK // 64, 8, 128), lambda i: (i, 0, 0)),
        pl.BlockSpec(memory_space=pltpu.SMEM),
    ],
    out_specs=pl.BlockSpec((1, TC_CBLK), lambda i: (0, i)),
    out_shape=jax.ShapeDtypeStruct((1, B), jnp.float32),
)


def kernel(x_1, x_2, idx_1, idx_2, W, table, scale):
    feat = _tc_feat(x_1, x_2, W)
    tbl_lin = _sc_lin(table.T).reshape(TROWS, EMBED_DIM)
    idx1 = idx_1.astype(jnp.int32).reshape(B // 1024, 8, 128)
    idx2 = idx_2.astype(jnp.int32).reshape(B // 1024, 8, 128)
    prod = _sc_prod(tbl_lin, idx1, idx2)
    scale2d = jnp.asarray(scale, jnp.float32).reshape(1, 1)
    out = _tc_comb(feat, prod, scale2d)
    return out.reshape(B, 1)
